# baseline (speedup 1.0000x reference)
import functools

import jax
import jax.numpy as jnp
from jax import lax
from jax.experimental import pallas as pl
from jax.experimental.pallas import tpu as pltpu


def _mlp_body(ct5_ref, f_ref, w15_ref, w1f_ref, w2_ref, b2_ref, w34_ref,
              b3_ref, outT_ref):
    f = f_ref[...].astype(jnp.bfloat16)
    pre = jnp.dot(f, w1f_ref[...], preferred_element_type=jnp.float32)
    pre = pre + lax.dot_general(
        ct5_ref[0].astype(jnp.bfloat16), w15_ref[...],
        (((0,), (0,)), ((), ())), preferred_element_type=jnp.float32)
    h = (pre * jax.nn.sigmoid(pre)).astype(jnp.bfloat16)
    pre2 = jnp.dot(h, w2_ref[...], preferred_element_type=jnp.float32)
    pre2 = pre2 + b2_ref[...]
    h2 = (pre2 * jax.nn.sigmoid(pre2)).astype(jnp.bfloat16)
    outT = lax.dot_general(w34_ref[...], h2, (((1,), (1,)), ((), ())),
                           preferred_element_type=jnp.float32)
    outT_ref[0] = outT + b3_ref[...]


@functools.partial(jax.jit, static_argnames=("blk",))
def _run(c_t, features, t, W1, b1, W2, b2, W3, b3, blk):
    batch, hidden = features.shape
    nblk = batch // blk
    ct5 = jnp.concatenate(
        [c_t.T, t.T, jnp.ones((1, batch), jnp.float32)], axis=0)
    ct5 = ct5.reshape(5, nblk, blk).transpose(1, 0, 2)
    w15 = jnp.concatenate(
        [W1[:, :3].T, W1[:, 3 + hidden:].T, b1.reshape(1, hidden)],
        axis=0).astype(jnp.bfloat16)
    w1f = W1[:, 3:3 + hidden].T.astype(jnp.bfloat16)
    w2 = W2.T.astype(jnp.bfloat16)
    b2r = b2.reshape(1, hidden)
    w34 = jnp.concatenate(
        [W3, jnp.zeros((1, hidden), jnp.float32)], axis=0
    ).astype(jnp.bfloat16)
    b3r = jnp.concatenate([b3, jnp.zeros((1,), jnp.float32)]).reshape(4, 1)

    grid = (nblk,)
    outT = pl.pallas_call(
        _mlp_body,
        grid=grid,
        in_specs=[
            pl.BlockSpec((1, 5, blk), lambda i: (i, 0, 0)),
            pl.BlockSpec((blk, hidden), lambda i: (i, 0)),
            pl.BlockSpec((5, hidden), lambda i: (0, 0)),
            pl.BlockSpec((hidden, hidden), lambda i: (0, 0)),
            pl.BlockSpec((hidden, hidden), lambda i: (0, 0)),
            pl.BlockSpec((1, hidden), lambda i: (0, 0)),
            pl.BlockSpec((4, hidden), lambda i: (0, 0)),
            pl.BlockSpec((4, 1), lambda i: (0, 0)),
        ],
        out_specs=pl.BlockSpec((1, 4, blk), lambda i: (i, 0, 0)),
        out_shape=jax.ShapeDtypeStruct((nblk, 4, blk), jnp.float32),
        compiler_params=pltpu.CompilerParams(
            dimension_semantics=("arbitrary",)),
    )(ct5, features, w15, w1f, w2, b2r, w34, b3r)
    return outT[:, :3, :].transpose(0, 2, 1).reshape(batch, 3)


def kernel(c_t, features, t, W1, b1, W2, b2, W3, b3):
    return _run(c_t, features, t, W1, b1, W2, b2, W3, b3, blk=2000)


# speedup vs baseline: 1.1162x; 1.1162x over previous
import functools

import jax
import jax.numpy as jnp
from jax import lax
from jax.experimental import pallas as pl
from jax.experimental.pallas import tpu as pltpu


def _mlp_body(ct5_ref, f_ref, w15_ref, w1f_ref, w2_ref, b2_ref, w34_ref,
              b3_ref, outT_ref):
    f = f_ref[...].astype(jnp.bfloat16)
    pre = jnp.dot(f, w1f_ref[...], preferred_element_type=jnp.float32)
    pre = pre + lax.dot_general(
        ct5_ref[0].astype(jnp.bfloat16), w15_ref[...],
        (((0,), (0,)), ((), ())), preferred_element_type=jnp.float32)
    h = (pre * jax.nn.sigmoid(pre)).astype(jnp.bfloat16)
    pre2 = jnp.dot(h, w2_ref[...], preferred_element_type=jnp.float32)
    pre2 = pre2 + b2_ref[...]
    h2 = (pre2 * jax.nn.sigmoid(pre2)).astype(jnp.bfloat16)
    outT = lax.dot_general(w34_ref[...], h2, (((1,), (1,)), ((), ())),
                           preferred_element_type=jnp.float32)
    outT_ref[0] = outT + b3_ref[...]


@functools.partial(jax.jit, static_argnames=("blk",))
def _run(c_t, features, t, W1, b1, W2, b2, W3, b3, blk):
    batch, hidden = features.shape
    nblk = batch // blk
    ct5 = jnp.concatenate(
        [c_t.T, t.T, jnp.ones((1, batch), jnp.float32)], axis=0)
    ct5 = ct5.reshape(5, nblk, blk).transpose(1, 0, 2)
    w15 = jnp.concatenate(
        [W1[:, :3].T, W1[:, 3 + hidden:].T, b1.reshape(1, hidden)],
        axis=0).astype(jnp.bfloat16)
    w1f = W1[:, 3:3 + hidden].T.astype(jnp.bfloat16)
    w2 = W2.T.astype(jnp.bfloat16)
    b2r = b2.reshape(1, hidden)
    w34 = jnp.concatenate(
        [W3, jnp.zeros((1, hidden), jnp.float32)], axis=0
    ).astype(jnp.bfloat16)
    b3r = jnp.concatenate([b3, jnp.zeros((1,), jnp.float32)]).reshape(4, 1)

    grid = (nblk,)
    outT = pl.pallas_call(
        _mlp_body,
        grid=grid,
        in_specs=[
            pl.BlockSpec((1, 5, blk), lambda i: (i, 0, 0)),
            pl.BlockSpec((blk, hidden), lambda i: (i, 0)),
            pl.BlockSpec((5, hidden), lambda i: (0, 0)),
            pl.BlockSpec((hidden, hidden), lambda i: (0, 0)),
            pl.BlockSpec((hidden, hidden), lambda i: (0, 0)),
            pl.BlockSpec((1, hidden), lambda i: (0, 0)),
            pl.BlockSpec((4, hidden), lambda i: (0, 0)),
            pl.BlockSpec((4, 1), lambda i: (0, 0)),
        ],
        out_specs=pl.BlockSpec((1, 4, blk), lambda i: (i, 0, 0)),
        out_shape=jax.ShapeDtypeStruct((nblk, 4, blk), jnp.float32),
        compiler_params=pltpu.CompilerParams(
            dimension_semantics=("arbitrary",)),
    )(ct5, features, w15, w1f, w2, b2r, w34, b3r)
    return outT[:, :3, :].transpose(0, 2, 1).reshape(batch, 3)


def kernel(c_t, features, t, W1, b1, W2, b2, W3, b3):
    return _run(c_t, features, t, W1, b1, W2, b2, W3, b3, blk=5000)


# speedup vs baseline: 1.1654x; 1.0441x over previous
import functools

import jax
import jax.numpy as jnp
from jax import lax
from jax.experimental import pallas as pl
from jax.experimental.pallas import tpu as pltpu


def _mlp_body(ct5_ref, f_ref, w15_ref, w1f_ref, w2_ref, b2_ref, w34_ref,
              b3_ref, outT_ref):
    f = f_ref[...].astype(jnp.bfloat16)
    pre = jnp.dot(f, w1f_ref[...], preferred_element_type=jnp.float32)
    pre = pre + lax.dot_general(
        ct5_ref[0].astype(jnp.bfloat16), w15_ref[...],
        (((0,), (0,)), ((), ())), preferred_element_type=jnp.float32)
    h = (pre * jax.nn.sigmoid(pre)).astype(jnp.bfloat16)
    pre2 = jnp.dot(h, w2_ref[...], preferred_element_type=jnp.float32)
    pre2 = pre2 + b2_ref[...]
    h2 = (pre2 * jax.nn.sigmoid(pre2)).astype(jnp.bfloat16)
    outT = lax.dot_general(w34_ref[...], h2, (((1,), (1,)), ((), ())),
                           preferred_element_type=jnp.float32)
    outT_ref[0] = outT + b3_ref[...]


@functools.partial(jax.jit, static_argnames=("blk",))
def _run(c_t, features, t, W1, b1, W2, b2, W3, b3, blk):
    batch, hidden = features.shape
    nblk = batch // blk
    ct5 = jnp.concatenate(
        [c_t.T, t.T, jnp.ones((1, batch), jnp.float32)], axis=0)
    ct5 = ct5.reshape(5, nblk, blk).transpose(1, 0, 2)
    w15 = jnp.concatenate(
        [W1[:, :3].T, W1[:, 3 + hidden:].T, b1.reshape(1, hidden)],
        axis=0).astype(jnp.bfloat16)
    w1f = W1[:, 3:3 + hidden].T.astype(jnp.bfloat16)
    w2 = W2.T.astype(jnp.bfloat16)
    b2r = b2.reshape(1, hidden)
    w34 = jnp.concatenate(
        [W3, jnp.zeros((1, hidden), jnp.float32)], axis=0
    ).astype(jnp.bfloat16)
    b3r = jnp.concatenate([b3, jnp.zeros((1,), jnp.float32)]).reshape(4, 1)

    grid = (nblk,)
    outT = pl.pallas_call(
        _mlp_body,
        grid=grid,
        in_specs=[
            pl.BlockSpec((1, 5, blk), lambda i: (i, 0, 0)),
            pl.BlockSpec((blk, hidden), lambda i: (i, 0)),
            pl.BlockSpec((5, hidden), lambda i: (0, 0)),
            pl.BlockSpec((hidden, hidden), lambda i: (0, 0)),
            pl.BlockSpec((hidden, hidden), lambda i: (0, 0)),
            pl.BlockSpec((1, hidden), lambda i: (0, 0)),
            pl.BlockSpec((4, hidden), lambda i: (0, 0)),
            pl.BlockSpec((4, 1), lambda i: (0, 0)),
        ],
        out_specs=pl.BlockSpec((1, 4, blk), lambda i: (i, 0, 0)),
        out_shape=jax.ShapeDtypeStruct((nblk, 4, blk), jnp.float32),
        compiler_params=pltpu.CompilerParams(
            dimension_semantics=("arbitrary",)),
    )(ct5, features, w15, w1f, w2, b2r, w34, b3r)
    return outT[:, :3, :].transpose(0, 2, 1).reshape(batch, 3)


def kernel(c_t, features, t, W1, b1, W2, b2, W3, b3):
    return _run(c_t, features, t, W1, b1, W2, b2, W3, b3, blk=10000)


# speedup vs baseline: 1.4165x; 1.2154x over previous
import functools

import jax
import jax.numpy as jnp
from jax import lax
from jax.experimental import pallas as pl
from jax.experimental.pallas import tpu as pltpu


def _mlp_body(ct5_ref, f_ref, w15_ref, w1f_ref, w2_ref, b2_ref, w34_ref,
              b3_ref, outT_ref):
    f = f_ref[...].astype(jnp.bfloat16)
    pre = jnp.dot(f, w1f_ref[...], preferred_element_type=jnp.float32)
    pre = pre + lax.dot_general(
        ct5_ref[0].astype(jnp.bfloat16), w15_ref[...],
        (((0,), (0,)), ((), ())), preferred_element_type=jnp.float32)
    pre = pre.astype(jnp.bfloat16)
    h = pre * jax.nn.sigmoid(pre)
    pre2 = jnp.dot(h, w2_ref[...], preferred_element_type=jnp.float32)
    pre2 = (pre2 + b2_ref[...]).astype(jnp.bfloat16)
    h2 = pre2 * jax.nn.sigmoid(pre2)
    outT = lax.dot_general(w34_ref[...], h2, (((1,), (1,)), ((), ())),
                           preferred_element_type=jnp.float32)
    outT_ref[0] = outT + b3_ref[...]


@functools.partial(jax.jit, static_argnames=("blk",))
def _run(c_t, features, t, W1, b1, W2, b2, W3, b3, blk):
    batch, hidden = features.shape
    nblk = batch // blk
    ct5 = jnp.concatenate(
        [c_t.T, t.T, jnp.ones((1, batch), jnp.float32)], axis=0)
    ct5 = ct5.reshape(5, nblk, blk).transpose(1, 0, 2)
    w15 = jnp.concatenate(
        [W1[:, :3].T, W1[:, 3 + hidden:].T, b1.reshape(1, hidden)],
        axis=0).astype(jnp.bfloat16)
    w1f = W1[:, 3:3 + hidden].T.astype(jnp.bfloat16)
    w2 = W2.T.astype(jnp.bfloat16)
    b2r = b2.reshape(1, hidden)
    w34 = jnp.concatenate(
        [W3, jnp.zeros((1, hidden), jnp.float32)], axis=0
    ).astype(jnp.bfloat16)
    b3r = jnp.concatenate([b3, jnp.zeros((1,), jnp.float32)]).reshape(4, 1)

    grid = (nblk,)
    outT = pl.pallas_call(
        _mlp_body,
        grid=grid,
        in_specs=[
            pl.BlockSpec((1, 5, blk), lambda i: (i, 0, 0)),
            pl.BlockSpec((blk, hidden), lambda i: (i, 0)),
            pl.BlockSpec((5, hidden), lambda i: (0, 0)),
            pl.BlockSpec((hidden, hidden), lambda i: (0, 0)),
            pl.BlockSpec((hidden, hidden), lambda i: (0, 0)),
            pl.BlockSpec((1, hidden), lambda i: (0, 0)),
            pl.BlockSpec((4, hidden), lambda i: (0, 0)),
            pl.BlockSpec((4, 1), lambda i: (0, 0)),
        ],
        out_specs=pl.BlockSpec((1, 4, blk), lambda i: (i, 0, 0)),
        out_shape=jax.ShapeDtypeStruct((nblk, 4, blk), jnp.float32),
        compiler_params=pltpu.CompilerParams(
            dimension_semantics=("arbitrary",)),
    )(ct5, features, w15, w1f, w2, b2r, w34, b3r)
    return outT[:, :3, :].transpose(0, 2, 1).reshape(batch, 3)


def kernel(c_t, features, t, W1, b1, W2, b2, W3, b3):
    return _run(c_t, features, t, W1, b1, W2, b2, W3, b3, blk=4000)


# speedup vs baseline: 1.4553x; 1.0274x over previous
import functools

import jax
import jax.numpy as jnp
from jax import lax
from jax.experimental import pallas as pl
from jax.experimental.pallas import tpu as pltpu


def _mlp_body(ct5_ref, f_ref, w15_ref, w1f_ref, w2_ref, b2_ref, w34_ref,
              b3_ref, outT_ref):
    f = f_ref[...].astype(jnp.bfloat16)
    pre = jnp.dot(f, w1f_ref[...], preferred_element_type=jnp.float32)
    pre = pre + lax.dot_general(
        ct5_ref[0].astype(jnp.bfloat16), w15_ref[...],
        (((0,), (0,)), ((), ())), preferred_element_type=jnp.float32)
    pre = pre.astype(jnp.bfloat16)
    h = pre * (jnp.tanh(pre * 0.5) * 0.5 + 0.5)
    pre2 = jnp.dot(h, w2_ref[...], preferred_element_type=jnp.float32)
    pre2 = (pre2 + b2_ref[...]).astype(jnp.bfloat16)
    h2 = pre2 * (jnp.tanh(pre2 * 0.5) * 0.5 + 0.5)
    outT = lax.dot_general(w34_ref[...], h2, (((1,), (1,)), ((), ())),
                           preferred_element_type=jnp.float32)
    outT_ref[0] = outT + b3_ref[...]


@functools.partial(jax.jit, static_argnames=("blk",))
def _run(c_t, features, t, W1, b1, W2, b2, W3, b3, blk):
    batch, hidden = features.shape
    nblk = batch // blk
    ct5 = jnp.concatenate(
        [c_t.T, t.T, jnp.ones((1, batch), jnp.float32)], axis=0)
    ct5 = ct5.reshape(5, nblk, blk).transpose(1, 0, 2)
    w15 = jnp.concatenate(
        [W1[:, :3].T, W1[:, 3 + hidden:].T, b1.reshape(1, hidden)],
        axis=0).astype(jnp.bfloat16)
    w1f = W1[:, 3:3 + hidden].T.astype(jnp.bfloat16)
    w2 = W2.T.astype(jnp.bfloat16)
    b2r = b2.reshape(1, hidden)
    w34 = jnp.concatenate(
        [W3, jnp.zeros((1, hidden), jnp.float32)], axis=0
    ).astype(jnp.bfloat16)
    b3r = jnp.concatenate([b3, jnp.zeros((1,), jnp.float32)]).reshape(4, 1)

    grid = (nblk,)
    outT = pl.pallas_call(
        _mlp_body,
        grid=grid,
        in_specs=[
            pl.BlockSpec((1, 5, blk), lambda i: (i, 0, 0)),
            pl.BlockSpec((blk, hidden), lambda i: (i, 0)),
            pl.BlockSpec((5, hidden), lambda i: (0, 0)),
            pl.BlockSpec((hidden, hidden), lambda i: (0, 0)),
            pl.BlockSpec((hidden, hidden), lambda i: (0, 0)),
            pl.BlockSpec((1, hidden), lambda i: (0, 0)),
            pl.BlockSpec((4, hidden), lambda i: (0, 0)),
            pl.BlockSpec((4, 1), lambda i: (0, 0)),
        ],
        out_specs=pl.BlockSpec((1, 4, blk), lambda i: (i, 0, 0)),
        out_shape=jax.ShapeDtypeStruct((nblk, 4, blk), jnp.float32),
        compiler_params=pltpu.CompilerParams(
            dimension_semantics=("arbitrary",)),
    )(ct5, features, w15, w1f, w2, b2r, w34, b3r)
    return outT[:, :3, :].transpose(0, 2, 1).reshape(batch, 3)


def kernel(c_t, features, t, W1, b1, W2, b2, W3, b3):
    return _run(c_t, features, t, W1, b1, W2, b2, W3, b3, blk=4000)
